# baseline (speedup 1.0000x reference)
import functools

import jax
import jax.numpy as jnp
from jax import lax
from jax.experimental import pallas as pl
from jax.experimental.pallas import tpu as pltpu
from jax.experimental.pallas import tpu_sc as plsc

VOCAB = 1000000
DIM = 64
BATCH = 16384
HIST = 20

NC = 2
NS = 16
NW = NC * NS
B_PER_W = BATCH // NW
LOOK_PER_W = B_PER_W * HIST
BCHUNK = 16
CH = BCHUNK * HIST
NCHUNK = B_PER_W // BCHUNK
NBUF = 4

_mesh = plsc.VectorSubcoreMesh(core_axis_name="c", subcore_axis_name="s")


@functools.partial(
    pl.kernel,
    mesh=_mesh,
    out_type=jax.ShapeDtypeStruct((BATCH * HIST, DIM), jnp.float32),
    scratch_types=[
        pltpu.VMEM((HIST, BATCH // NW), jnp.int32),
        pltpu.VMEM((LOOK_PER_W,), jnp.int32),
        pltpu.VMEM((NBUF, CH, DIM), jnp.float32),
    ] + [pltpu.SemaphoreType.DMA] * (2 * NBUF),
    compiler_params=pltpu.CompilerParams(
        use_tc_tiling_on_sc=False, needs_layout_passes=False),
)
def _embed(table, xt, out, idx_raw, idx_bm, rows, *sems):
    gsem = sems[:NBUF]
    ssem = sems[NBUF:]
    wid = lax.axis_index("s") * NC + lax.axis_index("c")
    b_base = wid * B_PER_W

    pltpu.sync_copy(xt.at[:, pl.ds(b_base, B_PER_W)], idx_raw)

    @pl.loop(0, LOOK_PER_W // 16)
    def _perm(i):
        p = lax.iota(jnp.int32, 16) + i * 16
        h = p % HIST
        b = p // HIST
        idx_bm[pl.ds(i * 16, 16)] = plsc.load_gather(idx_raw, [h, b])

    def start_gather(c, b):
        pltpu.async_copy(
            table.at[idx_bm.at[pl.ds(c * CH, CH)]], rows.at[b], gsem[b])

    for b in range(NBUF):
        start_gather(b, b)

    def wait_writeback(b):
        pltpu.make_async_copy(
            rows.at[b], out.at[pl.ds(0, CH)], ssem[b]).wait()

    @pl.loop(0, NCHUNK, step=NBUF)
    def _group(g):
        for b in range(NBUF):
            c = g + b
            pltpu.make_async_copy(
                table.at[pl.ds(0, CH)], rows.at[b], gsem[b]).wait()
            pltpu.async_copy(
                rows.at[b], out.at[pl.ds((b_base + c * BCHUNK) * HIST, CH)],
                ssem[b])
        for b in range(NBUF):
            cn = g + b + NBUF

            @pl.when(cn < NCHUNK)
            def _(b=b, cn=cn):
                wait_writeback(b)
                start_gather(cn, b)

    for b in range(NBUF):
        wait_writeback(b)


def kernel(x, weight):
    xt = x.astype(jnp.int32).T
    return _embed(weight, xt).reshape(BATCH, HIST, DIM)


# speedup vs baseline: 1.0022x; 1.0022x over previous
import functools

import jax
import jax.numpy as jnp
from jax import lax
from jax.experimental import pallas as pl
from jax.experimental.pallas import tpu as pltpu
from jax.experimental.pallas import tpu_sc as plsc

VOCAB = 1000000
DIM = 64
BATCH = 16384
HIST = 20

NC = 2
NS = 16
NW = NC * NS
NLOOK = BATCH * HIST
LOOK_PER_W = NLOOK // NW
CH = 320
NCHUNK = LOOK_PER_W // CH
NBUF = 4

_mesh = plsc.VectorSubcoreMesh(core_axis_name="c", subcore_axis_name="s")


@functools.partial(
    pl.kernel,
    mesh=_mesh,
    out_type=jax.ShapeDtypeStruct((NLOOK, DIM), jnp.float32),
    scratch_types=[
        pltpu.VMEM((LOOK_PER_W,), jnp.int32),
        pltpu.VMEM((NBUF, CH, DIM), jnp.float32),
    ] + [pltpu.SemaphoreType.DMA] * (2 * NBUF),
    compiler_params=pltpu.CompilerParams(
        use_tc_tiling_on_sc=False, needs_layout_passes=False),
)
def _embed(table, idx, out, idx_v, rows, *sems):
    gsem = sems[:NBUF]
    ssem = sems[NBUF:]
    wid = lax.axis_index("s") * NC + lax.axis_index("c")
    base = wid * LOOK_PER_W

    pltpu.sync_copy(idx.at[pl.ds(base, LOOK_PER_W)], idx_v)

    def start_gather(c, b):
        pltpu.async_copy(
            table.at[idx_v.at[pl.ds(c * CH, CH)]], rows.at[b], gsem[b])

    def wait_writeback(b):
        pltpu.make_async_copy(
            rows.at[b], out.at[pl.ds(0, CH)], ssem[b]).wait()

    for b in range(NBUF):
        start_gather(b, b)

    @pl.loop(0, NCHUNK, step=NBUF)
    def _group(g):
        for b in range(NBUF):
            c = g + b
            pltpu.make_async_copy(
                table.at[pl.ds(0, CH)], rows.at[b], gsem[b]).wait()
            pltpu.async_copy(
                rows.at[b], out.at[pl.ds(base + c * CH, CH)], ssem[b])
        for b in range(NBUF):
            cn = g + b + NBUF

            @pl.when(cn < NCHUNK)
            def _(b=b, cn=cn):
                wait_writeback(b)
                start_gather(cn, b)

    for b in range(NBUF):
        wait_writeback(b)


def kernel(x, weight):
    idx = x.astype(jnp.int32).reshape(-1)
    return _embed(weight, idx).reshape(BATCH, HIST, DIM)


# speedup vs baseline: 1.0054x; 1.0031x over previous
import functools

import jax
import jax.numpy as jnp
from jax import lax
from jax.experimental import pallas as pl
from jax.experimental.pallas import tpu as pltpu
from jax.experimental.pallas import tpu_sc as plsc

VOCAB = 1000000
DIM = 64
BATCH = 16384
HIST = 20

NC = 2
NS = 16
NW = NC * NS
NLOOK = BATCH * HIST
LOOK_PER_W = NLOOK // NW
CH = 160
NCHUNK = LOOK_PER_W // CH
NBUF = 4

_mesh = plsc.VectorSubcoreMesh(core_axis_name="c", subcore_axis_name="s")


@functools.partial(
    pl.kernel,
    mesh=_mesh,
    out_type=jax.ShapeDtypeStruct((NLOOK, DIM), jnp.float32),
    scratch_types=[
        pltpu.VMEM((LOOK_PER_W,), jnp.int32),
        pltpu.VMEM((NBUF, CH, 2 * DIM), jnp.float32),
    ] + [pltpu.SemaphoreType.DMA] * (2 * NBUF),
    compiler_params=pltpu.CompilerParams(
        use_tc_tiling_on_sc=False, needs_layout_passes=False),
)
def _embed(table, idx, out, idx_v, rows, *sems):
    gsem = sems[:NBUF]
    ssem = sems[NBUF:]
    wid = lax.axis_index("s") * NC + lax.axis_index("c")
    base = wid * LOOK_PER_W

    pltpu.sync_copy(idx.at[pl.ds(base, LOOK_PER_W)], idx_v)

    def start_gather(c, b):
        pltpu.async_copy(
            table.at[idx_v.at[pl.ds(c * CH, CH)]], rows.at[b], gsem[b])

    def wait_writeback(b):
        pltpu.make_async_copy(
            rows.at[b].at[:, pl.ds(0, DIM)], out.at[pl.ds(0, CH)],
            ssem[b]).wait()

    for b in range(NBUF):
        start_gather(b, b)

    @pl.loop(0, NCHUNK, step=NBUF)
    def _group(g):
        for b in range(NBUF):
            c = g + b
            pltpu.make_async_copy(
                table.at[pl.ds(0, CH)], rows.at[b], gsem[b]).wait()
            pltpu.async_copy(
                rows.at[b].at[:, pl.ds(0, DIM)],
                out.at[pl.ds(base + c * CH, CH)], ssem[b])
        for b in range(NBUF):
            cn = g + b + NBUF

            @pl.when(cn < NCHUNK)
            def _(b=b, cn=cn):
                wait_writeback(b)
                start_gather(cn, b)

    for b in range(NBUF):
        wait_writeback(b)


def kernel(x, weight):
    idx = x.astype(jnp.int32).reshape(-1)
    table = jnp.pad(weight, ((0, 0), (0, DIM)))
    return _embed(table, idx).reshape(BATCH, HIST, DIM)


# speedup vs baseline: 1.0767x; 1.0709x over previous
import functools

import jax
import jax.numpy as jnp
from jax import lax
from jax.experimental import pallas as pl
from jax.experimental.pallas import tpu as pltpu
from jax.experimental.pallas import tpu_sc as plsc

VOCAB = 1000000
DIM = 64
BATCH = 16384
HIST = 20

NC = 2
NS = 16
NW = NC * NS
NLOOK = BATCH * HIST
LOOK_PER_W = NLOOK // NW
CH = 320
NCHUNK = LOOK_PER_W // CH
NBUF = 4

_mesh = plsc.VectorSubcoreMesh(core_axis_name="c", subcore_axis_name="s")


@functools.partial(
    pl.kernel,
    mesh=_mesh,
    out_type=jax.ShapeDtypeStruct((NLOOK, DIM), jnp.float32),
    scratch_types=[
        pltpu.VMEM((LOOK_PER_W,), jnp.int32),
        pltpu.VMEM((LOOK_PER_W,), jnp.int32),
        pltpu.VMEM((NBUF, CH, DIM), jnp.float32),
    ] + [pltpu.SemaphoreType.DMA] * (2 * NBUF),
    compiler_params=pltpu.CompilerParams(
        use_tc_tiling_on_sc=False, needs_layout_passes=False),
)
def _embed(table, idx, out, idx_v, idx2_v, rows, *sems):
    gsem = sems[:NBUF]
    ssem = sems[NBUF:]
    wid = lax.axis_index("s") * NC + lax.axis_index("c")
    base = wid * LOOK_PER_W

    pltpu.sync_copy(idx.at[pl.ds(base, LOOK_PER_W)], idx_v)

    @pl.loop(0, LOOK_PER_W // 16)
    def _dbl(i):
        idx2_v[pl.ds(i * 16, 16)] = idx_v[pl.ds(i * 16, 16)] * 2

    def start_gather(c, b):
        pltpu.async_copy(
            table.at[idx2_v.at[pl.ds(c * CH, CH)]], rows.at[b], gsem[b])

    def wait_writeback(b):
        pltpu.make_async_copy(
            rows.at[b], out.at[pl.ds(0, CH)], ssem[b]).wait()

    for b in range(NBUF):
        start_gather(b, b)

    @pl.loop(0, NCHUNK, step=NBUF)
    def _group(g):
        for b in range(NBUF):
            c = g + b
            pltpu.make_async_copy(
                table.at[pl.ds(0, CH)], rows.at[b], gsem[b]).wait()
            pltpu.async_copy(
                rows.at[b], out.at[pl.ds(base + c * CH, CH)], ssem[b])
        for b in range(NBUF):
            cn = g + b + NBUF

            @pl.when(cn < NCHUNK)
            def _(b=b, cn=cn):
                wait_writeback(b)
                start_gather(cn, b)

    for b in range(NBUF):
        wait_writeback(b)


def kernel(x, weight):
    idx = x.astype(jnp.int32).reshape(-1)
    wide = jnp.pad(weight, ((0, 0), (0, DIM)))
    table = wide.reshape(2 * VOCAB, DIM)
    return _embed(table, idx).reshape(BATCH, HIST, DIM)


# speedup vs baseline: 1.0785x; 1.0016x over previous
import functools

import jax
import jax.numpy as jnp
from jax import lax
from jax.experimental import pallas as pl
from jax.experimental.pallas import tpu as pltpu
from jax.experimental.pallas import tpu_sc as plsc

VOCAB = 1000000
DIM = 64
BATCH = 16384
HIST = 20

NC = 2
NS = 16
NW = NC * NS
NLOOK = BATCH * HIST
LOOK_PER_W = NLOOK // NW
CH = 320
NCHUNK = LOOK_PER_W // CH
NBUF = 4

_mesh = plsc.VectorSubcoreMesh(core_axis_name="c", subcore_axis_name="s")


@functools.partial(
    pl.kernel,
    mesh=_mesh,
    out_type=jax.ShapeDtypeStruct((NLOOK, DIM), jnp.float32),
    scratch_types=[
        pltpu.VMEM((LOOK_PER_W,), jnp.int32),
        pltpu.VMEM((NBUF, CH, DIM), jnp.float32),
    ] + [pltpu.SemaphoreType.DMA] * (2 * NBUF),
    compiler_params=pltpu.CompilerParams(
        use_tc_tiling_on_sc=False, needs_layout_passes=False),
)
def _embed(table, idx, out, idx_v, rows, *sems):
    gsem = sems[:NBUF]
    ssem = sems[NBUF:]
    wid = lax.axis_index("s") * NC + lax.axis_index("c")
    base = wid * LOOK_PER_W

    pltpu.sync_copy(idx.at[pl.ds(base, LOOK_PER_W)], idx_v)

    def start_gather(c, b):
        pltpu.async_copy(
            table.at[idx_v.at[pl.ds(c * CH, CH)]], rows.at[b], gsem[b])

    def wait_writeback(b):
        pltpu.make_async_copy(
            rows.at[b], out.at[pl.ds(0, CH)], ssem[b]).wait()

    for b in range(NBUF):
        start_gather(b, b)

    @pl.loop(0, NCHUNK, step=NBUF)
    def _group(g):
        for b in range(NBUF):
            c = g + b
            pltpu.make_async_copy(
                table.at[pl.ds(0, CH)], rows.at[b], gsem[b]).wait()
            pltpu.async_copy(
                rows.at[b], out.at[pl.ds(base + c * CH, CH)], ssem[b])
        for b in range(NBUF):
            cn = g + b + NBUF

            @pl.when(cn < NCHUNK)
            def _(b=b, cn=cn):
                wait_writeback(b)
                start_gather(cn, b)

    for b in range(NBUF):
        wait_writeback(b)


def kernel(x, weight):
    idx = (x.astype(jnp.int32) * 2).reshape(-1)
    wide = jnp.pad(weight, ((0, 0), (0, DIM)))
    table = wide.reshape(2 * VOCAB, DIM)
    return _embed(table, idx).reshape(BATCH, HIST, DIM)
